# baseline (speedup 1.0000x reference)
import jax
import jax.numpy as jnp
from jax.experimental import pallas as pl

_DIM = 128


def _body(off_ref, nxt_ref, table_ref, w_ref, out_ref):
    counts = jnp.maximum(nxt_ref[...] - off_ref[...], 0).astype(jnp.float32)
    v = jax.lax.dot_general(
        table_ref[...], w_ref[...],
        dimension_numbers=(((1,), (1,)), ((), ())),
        preferred_element_type=jnp.float32)
    out_ref[...] = counts * v


def kernel(input, offset, table, W):
    n = input.shape[0]
    b = offset.shape[0]
    blk = 512
    off = offset.reshape(b, 1)
    nxt = jnp.concatenate(
        [offset[1:], jnp.full((1,), n, offset.dtype)]).reshape(b, 1)
    return pl.pallas_call(
        _body,
        grid=(b // blk,),
        in_specs=[
            pl.BlockSpec((blk, 1), lambda i: (i, 0)),
            pl.BlockSpec((blk, 1), lambda i: (i, 0)),
            pl.BlockSpec((1, _DIM), lambda i: (0, 0)),
            pl.BlockSpec((_DIM, _DIM), lambda i: (0, 0)),
        ],
        out_specs=pl.BlockSpec((blk, _DIM), lambda i: (i, 0)),
        out_shape=jax.ShapeDtypeStruct((b, _DIM), jnp.float32),
    )(off, nxt, table, W)


# speedup vs baseline: 1.3318x; 1.3318x over previous
import jax
import jax.numpy as jnp
from jax.experimental import pallas as pl

_DIM = 128


def _body(off_ref, nxt_ref, table_ref, w_ref, out_ref):
    counts = jnp.maximum(nxt_ref[...] - off_ref[...], 0).astype(jnp.float32)
    v = jax.lax.dot_general(
        table_ref[...], w_ref[...],
        dimension_numbers=(((1,), (1,)), ((), ())),
        preferred_element_type=jnp.float32)
    out_ref[...] = jax.lax.dot_general(
        counts, v,
        dimension_numbers=(((1,), (0,)), ((), ())),
        preferred_element_type=jnp.float32)


def kernel(input, offset, table, W):
    n = input.shape[0]
    b = offset.shape[0]
    off = offset.reshape(b, 1)
    nxt = jnp.concatenate(
        [offset[1:], jnp.full((1,), n, offset.dtype)]).reshape(b, 1)
    return pl.pallas_call(
        _body,
        out_shape=jax.ShapeDtypeStruct((b, _DIM), jnp.float32),
    )(off, nxt, table, W)


# speedup vs baseline: 1.9048x; 1.4302x over previous
import functools

import jax
import jax.numpy as jnp
from jax.experimental import pallas as pl

_DIM = 128


def _body(n, off_ref, table_ref, w_ref, out_ref):
    b = off_ref.shape[0]
    o = off_ref[...]
    rolled = jnp.roll(o, -1, axis=0)
    row = jax.lax.broadcasted_iota(jnp.int32, (b, 1), 0)
    nxt = jnp.where(row == b - 1, n, rolled)
    counts = jnp.maximum(nxt - o, 0).astype(jnp.float32)
    v = jax.lax.dot_general(
        table_ref[...], w_ref[...],
        dimension_numbers=(((1,), (1,)), ((), ())),
        preferred_element_type=jnp.float32)
    out_ref[...] = counts * v


def kernel(input, offset, table, W):
    n = input.shape[0]
    b = offset.shape[0]
    return pl.pallas_call(
        functools.partial(_body, n),
        out_shape=jax.ShapeDtypeStruct((b, _DIM), jnp.float32),
    )(offset.reshape(b, 1), table, W)


# speedup vs baseline: 4.1571x; 2.1825x over previous
import functools

import jax
import jax.numpy as jnp
from jax.experimental import pallas as pl

_DIM = 128


def _body(n, off_ref, table_ref, w_ref, out_ref):
    r, c = off_ref.shape
    o = off_ref[...]
    lane_rot = jnp.roll(o, -1, axis=1)
    next_row = jnp.roll(lane_rot, -1, axis=0)
    lane = jax.lax.broadcasted_iota(jnp.int32, (r, c), 1)
    row = jax.lax.broadcasted_iota(jnp.int32, (r, c), 0)
    nxt = jnp.where(lane == c - 1, next_row, lane_rot)
    nxt = jnp.where((lane == c - 1) & (row == r - 1), n, nxt)
    counts = jnp.maximum(nxt - o, 0).astype(jnp.float32)
    v = jax.lax.dot_general(
        table_ref[...], w_ref[...],
        dimension_numbers=(((1,), (1,)), ((), ())),
        preferred_element_type=jnp.float32)
    out_ref[...] = counts[:, :, None] * v[None, :, :]


def kernel(input, offset, table, W):
    n = input.shape[0]
    b = offset.shape[0]
    r = b // _DIM
    out3 = pl.pallas_call(
        functools.partial(_body, n),
        out_shape=jax.ShapeDtypeStruct((r, _DIM, _DIM), jnp.float32),
    )(offset.reshape(r, _DIM), table, W)
    return out3.reshape(b, _DIM)


# speedup vs baseline: 4.2899x; 1.0320x over previous
import functools

import jax
import jax.numpy as jnp
from jax.experimental import pallas as pl

_DIM = 128


def _body(n, steps, off_ref, seam_ref, table_ref, w_ref, out_ref):
    blk, c = off_ref.shape
    o = off_ref[...]
    lane_rot = jnp.roll(o, -1, axis=1)
    next_row = jnp.roll(lane_rot, -1, axis=0)
    lane = jax.lax.broadcasted_iota(jnp.int32, (blk, c), 1)
    row = jax.lax.broadcasted_iota(jnp.int32, (blk, c), 0)
    nxt = jnp.where(lane == c - 1, next_row, lane_rot)
    i = pl.program_id(0)
    seam = seam_ref[...]
    fix = jnp.where(i == steps - 1, jnp.int32(n), seam[0, :, 0:1])
    nxt = jnp.where((lane == c - 1) & (row == blk - 1), fix, nxt)
    counts = jnp.maximum(nxt - o, 0).astype(jnp.float32)
    v = jax.lax.dot_general(
        table_ref[...], w_ref[...],
        dimension_numbers=(((1,), (1,)), ((), ())),
        preferred_element_type=jnp.float32)
    out_ref[...] = counts[:, :, None] * v[None, :, :]


def kernel(input, offset, table, W):
    n = input.shape[0]
    b = offset.shape[0]
    r = b // _DIM
    steps = 2
    blk = r // steps
    out3 = pl.pallas_call(
        functools.partial(_body, n, steps),
        grid=(steps,),
        in_specs=[
            pl.BlockSpec((blk, _DIM), lambda i: (i, 0)),
            pl.BlockSpec((1, 1, _DIM),
                         lambda i: (jnp.minimum((i + 1) * blk, r - 1), 0, 0)),
            pl.BlockSpec((1, _DIM), lambda i: (0, 0)),
            pl.BlockSpec((_DIM, _DIM), lambda i: (0, 0)),
        ],
        out_specs=pl.BlockSpec((blk, _DIM, _DIM), lambda i: (i, 0, 0)),
        out_shape=jax.ShapeDtypeStruct((r, _DIM, _DIM), jnp.float32),
    )(offset.reshape(r, _DIM), offset.reshape(r, 1, _DIM), table, W)
    return out3.reshape(b, _DIM)
